# baseline (speedup 1.0000x reference)
import functools

import jax
import jax.numpy as jnp
from jax import lax
from jax.experimental import pallas as pl
from jax.experimental.pallas import tpu as pltpu
from jax.experimental.pallas import tpu_sc as plsc

_NC = 2
_NS = 16
_NW = _NC * _NS


def _sc_gather_packed(flat_ids, table, chunk_rows):
    n = flat_ids.shape[0]
    e = table.shape[1]
    n2 = n // 2
    per_w = n2 // _NW
    n_chunks = per_w // chunk_rows

    mesh = plsc.VectorSubcoreMesh(
        core_axis_name="c", subcore_axis_name="s", num_cores=_NC, num_subcores=_NS
    )

    @functools.partial(
        pl.kernel,
        out_type=jax.ShapeDtypeStruct((n2, 2 * e), jnp.float32),
        mesh=mesh,
        scratch_types=[
            pltpu.VMEM((chunk_rows,), jnp.int32),
            pltpu.VMEM((chunk_rows,), jnp.int32),
            pltpu.VMEM((chunk_rows, e), jnp.float32),
            pltpu.VMEM((chunk_rows, e), jnp.float32),
            pltpu.SemaphoreType.DMA,
        ],
        compiler_params=pltpu.CompilerParams(use_tc_tiling_on_sc=False),
    )
    def gather_kernel(ids_hbm, table_hbm, out_hbm, idx_l, idx_r, lv, rv, sem):
        wid = lax.axis_index("s") * _NC + lax.axis_index("c")
        base = wid * per_w

        def body(i, carry):
            off = pl.multiple_of(base + i * chunk_rows, 8)
            pltpu.sync_copy(ids_hbm.at[pl.ds(off, chunk_rows)], idx_l)
            pltpu.sync_copy(ids_hbm.at[pl.ds(n2 + off, chunk_rows)], idx_r)
            cl = pltpu.async_copy(table_hbm.at[idx_l], lv, sem)
            cr = pltpu.async_copy(table_hbm.at[idx_r], rv, sem)
            cl.wait()
            cr.wait()
            pltpu.sync_copy(lv, out_hbm.at[pl.ds(off, chunk_rows), pl.ds(0, e)])
            pltpu.sync_copy(rv, out_hbm.at[pl.ds(off, chunk_rows), pl.ds(e, e)])
            return carry

        lax.fori_loop(0, n_chunks, body, 0)

    return gather_kernel(flat_ids, table)


def _tc_add_ln(xp, pos_tiled, gamma, beta, b, l, e, bb):
    b2 = b // 2
    rb = bb * l

    def body(x_ref, pos_ref, g_ref, b_ref, o_ref):
        x = x_ref[...] + pos_ref[...]
        g = g_ref[...]
        bt = b_ref[...]
        for h in range(2):
            xh = x[:, h * e:(h + 1) * e]
            mean = jnp.mean(xh, axis=-1, keepdims=True)
            c = xh - mean
            var = jnp.mean(c * c, axis=-1, keepdims=True)
            inv = lax.rsqrt(var + 1e-5)
            o_ref[h] = (c * inv * g + bt).reshape(bb, l, e)

    return pl.pallas_call(
        body,
        grid=(b2 // bb,),
        in_specs=[
            pl.BlockSpec((rb, 2 * e), lambda i: (i, 0)),
            pl.BlockSpec((rb, 2 * e), lambda i: (0, 0)),
            pl.BlockSpec((1, e), lambda i: (0, 0)),
            pl.BlockSpec((1, e), lambda i: (0, 0)),
        ],
        out_specs=pl.BlockSpec((2, bb, l, e), lambda i: (0, i, 0, 0)),
        out_shape=jax.ShapeDtypeStruct((2, b2, l, e), jnp.float32),
        compiler_params=pltpu.CompilerParams(vmem_limit_bytes=100 * 1024 * 1024),
    )(xp, pos_tiled, gamma.reshape(1, e), beta.reshape(1, e))


def kernel(input_ids, token_type_ids, word_table, pos_table, type_table, ln_gamma, ln_beta):
    b, l = input_ids.shape
    e = word_table.shape[1]
    bb = 64
    ctable = jnp.concatenate(
        [word_table + type_table[0], word_table + type_table[1]], axis=1
    ).reshape(-1, e)
    cids = (input_ids * 2 + token_type_ids).reshape(b * l)
    xp = _sc_gather_packed(cids, ctable, chunk_rows=800)
    pos_tiled = jnp.tile(pos_table[:l], (bb, 2))
    out = _tc_add_ln(xp, pos_tiled, ln_gamma, ln_beta, b, l, e, bb)
    return out.reshape(b, l, e)


# speedup vs baseline: 1.0096x; 1.0096x over previous
import functools

import jax
import jax.numpy as jnp
from jax import lax
from jax.experimental import pallas as pl
from jax.experimental.pallas import tpu as pltpu
from jax.experimental.pallas import tpu_sc as plsc

_NC = 2
_NS = 16
_NW = _NC * _NS


def _sc_gather_packed(flat_ids, table, chunk_rows):
    n = flat_ids.shape[0]
    e = table.shape[1]
    n2 = n // 2
    per_w = n2 // _NW
    n_chunks = per_w // chunk_rows

    mesh = plsc.VectorSubcoreMesh(
        core_axis_name="c", subcore_axis_name="s", num_cores=_NC, num_subcores=_NS
    )

    @functools.partial(
        pl.kernel,
        out_type=jax.ShapeDtypeStruct((n2, 2 * e), jnp.float32),
        mesh=mesh,
        scratch_types=[
            pltpu.VMEM((chunk_rows,), jnp.int32),
            pltpu.VMEM((chunk_rows,), jnp.int32),
            pltpu.VMEM((chunk_rows, e), jnp.float32),
            pltpu.VMEM((chunk_rows, e), jnp.float32),
            pltpu.SemaphoreType.DMA,
        ],
        compiler_params=pltpu.CompilerParams(use_tc_tiling_on_sc=False),
    )
    def gather_kernel(ids_hbm, table_hbm, out_hbm, idx_l, idx_r, lv, rv, sem):
        wid = lax.axis_index("s") * _NC + lax.axis_index("c")
        base = wid * per_w

        def body(i, carry):
            off = pl.multiple_of(base + i * chunk_rows, 8)
            pltpu.sync_copy(ids_hbm.at[pl.ds(off, chunk_rows)], idx_l)
            pltpu.sync_copy(ids_hbm.at[pl.ds(n2 + off, chunk_rows)], idx_r)
            cl = pltpu.async_copy(table_hbm.at[idx_l], lv, sem)
            cr = pltpu.async_copy(table_hbm.at[idx_r], rv, sem)
            cl.wait()
            cr.wait()
            pltpu.sync_copy(lv, out_hbm.at[pl.ds(off, chunk_rows), pl.ds(0, e)])
            pltpu.sync_copy(rv, out_hbm.at[pl.ds(off, chunk_rows), pl.ds(e, e)])
            return carry

        lax.fori_loop(0, n_chunks, body, 0)

    return gather_kernel(flat_ids, table)


def _tc_add_ln(xp, pos_tiled, gamma, beta, b, l, e, bb):
    b2 = b // 2
    rb = bb * l

    def body(x_ref, pos_ref, g_ref, b_ref, o_ref):
        x = x_ref[...] + pos_ref[...]
        g = g_ref[...]
        bt = b_ref[...]
        for h in range(2):
            xh = x[:, h * e:(h + 1) * e]
            mean = jnp.mean(xh, axis=-1, keepdims=True)
            c = xh - mean
            var = jnp.mean(c * c, axis=-1, keepdims=True)
            inv = lax.rsqrt(var + 1e-5)
            o_ref[h] = (c * inv * g + bt).reshape(bb, l, e)

    return pl.pallas_call(
        body,
        grid=(b2 // bb,),
        in_specs=[
            pl.BlockSpec((rb, 2 * e), lambda i: (i, 0)),
            pl.BlockSpec((rb, 2 * e), lambda i: (0, 0)),
            pl.BlockSpec((1, e), lambda i: (0, 0)),
            pl.BlockSpec((1, e), lambda i: (0, 0)),
        ],
        out_specs=pl.BlockSpec((2, bb, l, e), lambda i: (0, i, 0, 0)),
        out_shape=jax.ShapeDtypeStruct((2, b2, l, e), jnp.float32),
        compiler_params=pltpu.CompilerParams(vmem_limit_bytes=100 * 1024 * 1024),
    )(xp, pos_tiled, gamma.reshape(1, e), beta.reshape(1, e))


def kernel(input_ids, token_type_ids, word_table, pos_table, type_table, ln_gamma, ln_beta):
    b, l = input_ids.shape
    e = word_table.shape[1]
    bb = 32
    ctable = jnp.concatenate(
        [word_table + type_table[0], word_table + type_table[1]], axis=1
    ).reshape(-1, e)
    cids = (input_ids * 2 + token_type_ids).reshape(b * l)
    xp = _sc_gather_packed(cids, ctable, chunk_rows=800)
    pos_tiled = jnp.tile(pos_table[:l], (bb, 2))
    out = _tc_add_ln(xp, pos_tiled, ln_gamma, ln_beta, b, l, e, bb)
    return out.reshape(b, l, e)
